# baseline (speedup 1.0000x reference)
import functools

import jax
import jax.numpy as jnp
from jax import lax
from jax.experimental import pallas as pl
from jax.experimental.pallas import tpu as pltpu
from jax.experimental.pallas import tpu_sc as plsc

CHUNK_ROWS = 16
NBUF = 4
LANES = 16


@functools.lru_cache(maxsize=None)
def _build_gather(batch: int, fields: int, dim: int, cr: int, nbuf: int):
    mesh = plsc.VectorSubcoreMesh(core_axis_name="c", subcore_axis_name="s")
    n_workers = mesh.num_cores * mesh.num_subcores
    assert batch % n_workers == 0
    rows_per_w = batch // n_workers
    assert rows_per_w % cr == 0
    n_chunks = rows_per_w // cr
    assert n_chunks % nbuf == 0
    chunk = cr * fields
    assert chunk % LANES == 0 and chunk % 8 == 0
    n_vecs = chunk // LANES

    @functools.partial(
        pl.kernel,
        out_type=jax.ShapeDtypeStruct((batch * fields, dim), jnp.float32),
        mesh=mesh,
        scratch_types=[
            [pltpu.VMEM((cr, fields), jnp.int32) for _ in range(nbuf)],
            [pltpu.VMEM((chunk,), jnp.int32) for _ in range(nbuf)],
            [pltpu.VMEM((chunk, dim), jnp.float32) for _ in range(nbuf)],
            [pltpu.SemaphoreType.DMA for _ in range(nbuf)],
            [pltpu.SemaphoreType.DMA for _ in range(nbuf)],
        ],
        compiler_params=pltpu.CompilerParams(use_tc_tiling_on_sc=False,
                                             needs_layout_passes=False),
    )
    def gather_kernel(idx_hbm, table_hbm, out_hbm,
                      blk_v, idx_v, rows_v, gsem, ssem):
        wid = lax.axis_index("s") * mesh.num_cores + lax.axis_index("c")
        base = wid * rows_per_w

        def stage(b, g):
            pltpu.sync_copy(idx_hbm.at[pl.ds(base + g * cr, cr), :],
                            blk_v[b])
            magic = -(-(1 << 16) // fields)
            lane = lax.iota(jnp.int32, LANES)
            for v in range(n_vecs):
                k = lane + v * LANES
                r = lax.shift_right_logical(k * magic, 16)
                c = k - r * fields
                idx_v[b][pl.ds(v * LANES, LANES)] = plsc.load_gather(
                    blk_v[b], [r, c])
            pltpu.async_copy(table_hbm.at[idx_v[b]], rows_v[b], gsem[b])

        for b in range(nbuf):
            stage(b, b)

        def body(grp, carry):
            g0 = grp * nbuf
            for b in range(nbuf):
                g = g0 + b
                out_slc = out_hbm.at[pl.ds((base + g * cr) * fields, chunk)]
                pltpu.make_async_copy(table_hbm.at[idx_v[b]], rows_v[b],
                                      gsem[b]).wait()
                pltpu.async_copy(rows_v[b], out_slc, ssem[b])

                @pl.when(g + nbuf < n_chunks)
                def _():
                    pltpu.make_async_copy(rows_v[b], out_slc, ssem[b]).wait()
                    stage(b, g + nbuf)

            return carry

        lax.fori_loop(0, n_chunks // nbuf, body, 0)

        for b in range(nbuf):
            g = n_chunks - nbuf + b
            out_slc = out_hbm.at[pl.ds((base + g * cr) * fields, chunk)]
            pltpu.make_async_copy(rows_v[b], out_slc, ssem[b]).wait()

    return gather_kernel


def kernel(input, weight):
    b, f = input.shape
    out = _build_gather(b, f, weight.shape[1], CHUNK_ROWS, NBUF)(
        input.astype(jnp.int32), weight)
    return out.reshape(b, f, weight.shape[1])
